# baseline (speedup 1.0000x reference)
import functools

import jax
import jax.numpy as jnp
from jax import lax
from jax.experimental import pallas as pl
from jax.experimental.pallas import tpu as pltpu
from jax.experimental.pallas import tpu_sc as plsc

N = 10000
E = 320000
D = 128

NC = 2
NS = 16
NW = NC * NS

K = 128
CH = 80
EW = CH * K
E_PAD = NW * EW
ROWS_PAD = E_PAD // K
TRASH = N
DEG_ROWS = 10240
DRPT = DEG_ROWS // NS
SPARE = 347136
BN_CAP = SPARE + NW * 16
TRREC = 383
ACC_R = 384
A_ROWS = ACC_R * NW
RANK1 = 1

_mesh = plsc.VectorSubcoreMesh(core_axis_name="c", subcore_axis_name="s")
_sc_params = pltpu.CompilerParams(needs_layout_passes=False)
_f32 = jnp.float32
_i32 = jnp.int32


@functools.partial(
    pl.kernel,
    out_type=[jax.ShapeDtypeStruct((NC * DEG_ROWS,), _f32),
              jax.ShapeDtypeStruct((NW, 1, 32), _f32)],
    mesh=_mesh,
    compiler_params=_sc_params,
    scratch_types=[
        pltpu.VMEM((CH, K), _i32),
        pltpu.VMEM((DRPT,), _f32),
        pltpu.VMEM((K,), _f32),
        pltpu.VMEM((1, 32), _f32),
        pltpu.VMEM_SHARED((DEG_ROWS,), _f32),
    ],
)
def _k1_deg(dst_hbm, deg_hbm, cnt_hbm, idx_v, zbuf, ones_v, cntv, acc):
    c = lax.axis_index("c")
    s = lax.axis_index("s")
    wid = s * NC + c
    for i in range(DRPT // 16):
        zbuf[pl.ds(i * 16, 16)] = jnp.zeros((16,), _f32)
    for i in range(K // 16):
        ones_v[pl.ds(i * 16, 16)] = jnp.ones((16,), _f32)
    cntv[0, pl.ds(0, 16)] = jnp.zeros((16,), _f32)
    cntv[0, pl.ds(16, 16)] = jnp.zeros((16,), _f32)
    pltpu.sync_copy(zbuf, acc.at[pl.ds(s * DRPT, DRPT)])
    plsc.subcore_barrier()
    pltpu.sync_copy(dst_hbm.at[pl.ds(wid * CH, CH)], idx_v)
    z16 = jnp.zeros((16,), _i32)

    def body(j, carry):
        pltpu.sync_copy(ones_v, acc.at[idx_v.at[j]], add=True)
        for l in range(K // 16):
            dv = idx_v[j, pl.ds(l * 16, 16)]
            bv = dv & 31
            rank, last = plsc.scan_count(bv)
            plsc.addupdate_scatter(
                cntv, [z16, bv],
                (rank + (1 - RANK1)).astype(_f32), mask=last)
        return carry

    lax.fori_loop(0, CH, body, 0)
    plsc.subcore_barrier()
    pltpu.sync_copy(acc.at[pl.ds(s * DRPT, DRPT)],
                    deg_hbm.at[pl.ds(c * DEG_ROWS + s * DRPT, DRPT)])
    pltpu.sync_copy(cntv, cnt_hbm.at[wid])


def _k2a_body(x_ref, degt_ref, w_ref, g_ref):
    deg = degt_ref[:, 0:1] + degt_ref[:, 1:2] + 1.0
    dinv = lax.rsqrt(deg)
    h = jnp.dot(x_ref[...], w_ref[...], preferred_element_type=_f32)
    g_ref[...] = h * dinv


def _k2a_scale(x, deg_t, w):
    blk = 1000
    return pl.pallas_call(
        _k2a_body,
        grid=(N // blk,),
        in_specs=[
            pl.BlockSpec((blk, D), lambda i: (i, 0)),
            pl.BlockSpec((blk, 2), lambda i: (i, 0)),
            pl.BlockSpec((D, D), lambda i: (0, 0)),
        ],
        out_specs=pl.BlockSpec((blk, D), lambda i: (i, 0)),
        out_shape=jax.ShapeDtypeStruct((N, D), _f32),
    )(x, deg_t, w)


def _k2b_body(cf_ref, ut_ref, mb_ref, ms_ref, pos_ref,
              offs_ref, bstart_ref, btot_ref):
    cf = jnp.floor((cf_ref[...] + 15.0) * (1.0 / 16.0)) * 16.0
    btot = jnp.dot(cf, mb_ref[...], preferred_element_type=_f32)
    rb = jnp.floor((btot + 127.0) * (1.0 / 128.0)) * 128.0
    row_inc = jnp.dot(rb, ut_ref[...], preferred_element_type=_f32)
    rt = row_inc[:, 127:128]
    rows_list = [jnp.zeros((1, 1), _f32)]
    run = rt[0:1, :]
    for i in range(1, 8):
        rows_list.append(run)
        run = run + rt[i:i + 1, :]
    row_base = jnp.concatenate(rows_list, axis=0)
    lane_excl = row_inc - rb + row_base
    bucket_base = (lane_excl - pos_ref[...] * rb) * (1.0 / 32.0)
    wex = jnp.dot(cf, ms_ref[...], preferred_element_type=_f32)
    offs_ref[...] = (bucket_base + wex).astype(_i32)
    bstart_ref[...] = bucket_base.astype(_i32)
    btot_ref[...] = btot.astype(_i32)


def _k2b_offsets(cf, ut, mb, ms, posm):
    spec = pl.BlockSpec((8, 128), lambda: (0, 0))
    mspec = pl.BlockSpec((D, D), lambda: (0, 0))
    return pl.pallas_call(
        _k2b_body,
        grid=(),
        in_specs=[spec, mspec, mspec, mspec, spec],
        out_specs=[spec, spec, spec],
        out_shape=[jax.ShapeDtypeStruct((8, 128), _i32)] * 3,
    )(cf, ut, mb, ms, posm)


@functools.partial(
    pl.kernel,
    out_type=jax.ShapeDtypeStruct((BN_CAP,), _i32),
    mesh=_mesh,
    compiler_params=_sc_params,
    scratch_types=[
        pltpu.VMEM((CH, K), _i32),
        pltpu.VMEM((CH, K), _i32),
        pltpu.VMEM((1, 32), _i32),
        pltpu.VMEM((2, K), _i32),
        pltpu.VMEM((2, K), _i32),
        pltpu.SemaphoreType.DMA,
        pltpu.SemaphoreType.DMA,
    ],
)
def _k3_route(src_hbm, dst_hbm, offs_hbm, bin_hbm,
              idx_s, idx_d, base, recb, posb, sem0, sem1):
    c = lax.axis_index("c")
    s = lax.axis_index("s")
    wid = s * NC + c
    pltpu.sync_copy(src_hbm.at[pl.ds(wid * CH, CH)], idx_s)
    pltpu.sync_copy(dst_hbm.at[pl.ds(wid * CH, CH)], idx_d)
    pltpu.sync_copy(offs_hbm.at[wid], base)
    z16 = jnp.zeros((16,), _i32)

    def fill_row(r, q):
        for l in range(K // 16):
            sv = idx_s[r, pl.ds(l * 16, 16)]
            dv = idx_d[r, pl.ds(l * 16, 16)]
            bv = dv & 31
            rec = (sv << 9) | (dv >> 5)
            rank, last = plsc.scan_count(bv)
            gb = plsc.load_gather(base, [z16, bv])
            pos = gb + rank - RANK1
            plsc.store_scatter(base, [z16, bv], pos + 1, mask=last)
            recb[q, pl.ds(l * 16, 16)] = rec
            posb[q, pl.ds(l * 16, 16)] = jnp.clip(pos, 0, BN_CAP - 1)

    def body(t, carry):
        r0 = t * 2
        fill_row(r0, 0)
        pltpu.sync_copy(recb.at[0], bin_hbm.at[posb.at[0]])
        fill_row(r0 + 1, 1)
        pltpu.sync_copy(recb.at[1], bin_hbm.at[posb.at[1]])
        return carry

    lax.fori_loop(0, CH // 2, body, 0)

    iota = lax.iota(_i32, 16)
    trv = jnp.full((16,), TRREC, _i32)
    for grp in range(4):
        for bi in range(8):
            b = grp * 8 + bi
            vec = base[0, pl.ds((b // 16) * 16, 16)]
            end = vec[b % 16]
            slack = (16 - (end & 15)) & 15
            posv = jnp.where(iota < slack, end + iota,
                             SPARE + wid * 16 + iota)
            posb[0, pl.ds(bi * 16, 16)] = jnp.clip(posv, 0, BN_CAP - 1)
            recb[0, pl.ds(bi * 16, 16)] = trv
        pltpu.sync_copy(recb.at[0], bin_hbm.at[posb.at[0]])


@functools.partial(
    pl.kernel,
    out_type=jax.ShapeDtypeStruct((A_ROWS, D), _f32),
    mesh=_mesh,
    compiler_params=_sc_params,
    scratch_types=[
        pltpu.VMEM((ACC_R, D), _f32),
        pltpu.VMEM((K, D), _f32),
        pltpu.VMEM((K, D), _f32),
        pltpu.VMEM((2, K), _i32),
        pltpu.VMEM((2, K), _i32),
        pltpu.VMEM((16, 16), _i32),
        pltpu.VMEM((8, 128), _i32),
        pltpu.VMEM((8, 128), _i32),
        pltpu.VMEM((2, K), _i32),
        pltpu.SemaphoreType.DMA,
        pltpu.SemaphoreType.DMA,
        pltpu.SemaphoreType.DMA,
        pltpu.SemaphoreType.DMA,
    ],
)
def _k4_consume(g_hbm, bin_hbm, bstart_hbm, btot_hbm, a_hbm,
                acc, rows0, rows1, recb, sidx, didx, bsv, btv, oidx,
                semg0, semg1, semr0, semr1):
    c = lax.axis_index("c")
    s = lax.axis_index("s")
    o = s * NC + c
    iota = lax.iota(_i32, 16)
    zv = jnp.zeros((16,), _f32)

    def zrow(i, carry):
        for j in range(D // 16):
            acc[i, pl.ds(j * 16, 16)] = zv
        return carry
    lax.fori_loop(0, ACC_R, zrow, 0)
    pltpu.sync_copy(bstart_hbm, bsv)
    pltpu.sync_copy(btot_hbm, btv)
    start = jnp.int32(0)
    n = jnp.int32(0)
    for b in range(32):
        svec = bsv[b // 4, pl.ds((b % 4) * 32, 16)]
        tvec = btv[b // 4, pl.ds((b % 4) * 32, 16)]
        start = lax.select(o == b, svec[0], start)
        n = lax.select(o == b, tvec[0], n)
    start = pl.multiple_of(start, K)
    nch = (n + jnp.int32(K - 1)) >> 7

    def load_rec(k2, q, sem):
        pltpu.async_copy(bin_hbm.at[pl.ds(start + k2 * K, K)],
                         recb.at[q], sem)

    def unpack(k2, q):
        for m in range(8):
            rv = recb[q, pl.ds(m * 16, 16)]
            gl = k2 * K + m * 16 + iota
            valid = gl < n
            srcv = jnp.clip(jnp.where(valid, rv >> 9, 0), 0, N - 1)
            dlv = jnp.minimum(jnp.where(valid, rv & 511, ACC_R - 1),
                              ACC_R - 1)
            sidx[q, pl.ds(m * 16, 16)] = srcv
            didx[q * 8 + m, pl.ds(0, 16)] = dlv

    def accumulate(q, rows):
        def sub(m, carry):
            dv = didx[q * 8 + m, pl.ds(0, 16)]
            for l in range(16):
                dl = dv[l]
                for j in range(D // 16):
                    plsc.addupdate(acc.at[dl, pl.ds(j * 16, 16)],
                                   rows[m * 16 + l, pl.ds(j * 16, 16)])
            return carry
        lax.fori_loop(0, 8, sub, 0)

    pltpu.sync_copy(bin_hbm.at[pl.ds(start, K)], recb.at[0])
    unpack(0, 0)
    pltpu.async_copy(g_hbm.at[sidx.at[0]], rows0, semg0)

    @pl.when(nch > 1)
    def _():
        load_rec(1, 1, semr1)

    def body(k2, carry):
        @pl.when(k2 % 2 == 0)
        def _():
            @pl.when(k2 + 1 < nch)
            def _():
                pltpu.make_async_copy(bin_hbm.at[pl.ds(start, K)],
                                      recb.at[1], semr1).wait()
                unpack(k2 + 1, 1)
                pltpu.async_copy(g_hbm.at[sidx.at[1]], rows1, semg1)

            pltpu.make_async_copy(g_hbm.at[sidx.at[0]], rows0, semg0).wait()

            @pl.when(k2 + 2 < nch)
            def _():
                load_rec(k2 + 2, 0, semr0)
            accumulate(0, rows0)

        @pl.when(k2 % 2 == 1)
        def _():
            @pl.when(k2 + 1 < nch)
            def _():
                pltpu.make_async_copy(bin_hbm.at[pl.ds(start, K)],
                                      recb.at[0], semr0).wait()
                unpack(k2 + 1, 0)
                pltpu.async_copy(g_hbm.at[sidx.at[0]], rows0, semg0)

            pltpu.make_async_copy(g_hbm.at[sidx.at[1]], rows1, semg1).wait()

            @pl.when(k2 + 2 < nch)
            def _():
                load_rec(k2 + 2, 1, semr1)
            accumulate(1, rows1)

        return carry

    lax.fori_loop(0, nch, body, 0)

    for t in range(ACC_R // K):
        for m in range(8):
            oidx[t % 2, pl.ds(m * 16, 16)] = (
                ((t * K + m * 16 + iota) << 5) | o)
        pltpu.sync_copy(acc.at[pl.ds(t * K, K)], a_hbm.at[oidx.at[t % 2]])


def _k5_body(a_ref, g_ref, degt_ref, b_ref, out_ref):
    deg = degt_ref[:, 0:1] + degt_ref[:, 1:2] + 1.0
    dinv = lax.rsqrt(deg)
    out_ref[...] = jnp.maximum(
        (a_ref[...] + g_ref[...]) * dinv + b_ref[...], 0.0)


def _k5_combine(a, g, deg_t, b2d):
    blk = 1000
    return pl.pallas_call(
        _k5_body,
        grid=(N // blk,),
        in_specs=[
            pl.BlockSpec((blk, D), lambda i: (i, 0)),
            pl.BlockSpec((blk, D), lambda i: (i, 0)),
            pl.BlockSpec((blk, 2), lambda i: (i, 0)),
            pl.BlockSpec((1, D), lambda i: (0, 0)),
        ],
        out_specs=pl.BlockSpec((blk, D), lambda i: (i, 0)),
        out_shape=jax.ShapeDtypeStruct((N, D), _f32),
    )(a, g, deg_t, b2d)


def kernel(x, edge_index, W, b):
    ei = edge_index.astype(_i32)
    pad = E_PAD - E
    src = jnp.concatenate([ei[0], jnp.zeros((pad,), _i32)])
    dst = jnp.concatenate([ei[1], jnp.full((pad,), TRASH, _i32)])
    src2d = src.reshape(ROWS_PAD, K)
    dst2d = dst.reshape(ROWS_PAD, K)

    deg_parts, counts = _k1_deg(dst2d)
    deg_t = jnp.transpose(deg_parts.reshape(NC, DEG_ROWS))[:N]
    g = _k2a_scale(x, deg_t, W)

    cf = counts.reshape(NW, 32).T.reshape(8, 128)
    ar = jnp.arange(128)
    blkid = ar // 32
    ut = jnp.triu(jnp.ones((128, 128), _f32))
    mb = (blkid[:, None] == blkid[None, :]).astype(_f32)
    ms = mb * (ar[:, None] < ar[None, :]).astype(_f32)
    posm = jnp.broadcast_to((ar % 32).astype(_f32), (8, 128))
    offs, bstart, btot = _k2b_offsets(cf, ut, mb, ms, posm)
    offs3d = offs.reshape(32, 32).T.reshape(NW, 1, 32)

    binned = _k3_route(src2d, dst2d, offs3d)
    a = _k4_consume(g, binned, bstart, btot)
    b2d = b.reshape(1, D)
    return _k5_combine(a, g, deg_t, b2d)


# speedup vs baseline: 2.8644x; 2.8644x over previous
import functools

import jax
import jax.numpy as jnp
from jax import lax
from jax.experimental import pallas as pl
from jax.experimental.pallas import tpu as pltpu
from jax.experimental.pallas import tpu_sc as plsc

N = 10000
E = 320000
D = 128

NC = 2
NS = 16
NW = NC * NS

K = 128
CH = 80
EW = CH * K
E_PAD = NW * EW
ROWS_PAD = E_PAD // K
TRASH = N
ACC_ROWS = 10240
RPT = ACC_ROWS // NS
GRP = 16

_mesh = plsc.VectorSubcoreMesh(core_axis_name="c", subcore_axis_name="s")


@functools.partial(
    pl.kernel,
    out_type=jax.ShapeDtypeStruct((NC * ACC_ROWS,), jnp.float32),
    mesh=_mesh,
    scratch_types=[
        pltpu.VMEM((CH, K), jnp.int32),
        pltpu.VMEM((RPT,), jnp.float32),
        pltpu.VMEM((K,), jnp.float32),
        pltpu.VMEM_SHARED((ACC_ROWS,), jnp.float32),
    ],
)
def _k1_deg(dst_hbm, out_hbm, idx_v, zbuf, ones_v, acc):
    c = lax.axis_index("c")
    s = lax.axis_index("s")
    wid = s * NC + c
    for i in range(RPT // 16):
        zbuf[pl.ds(i * 16, 16)] = jnp.zeros((16,), jnp.float32)
    for i in range(K // 16):
        ones_v[pl.ds(i * 16, 16)] = jnp.ones((16,), jnp.float32)
    pltpu.sync_copy(zbuf, acc.at[pl.ds(s * RPT, RPT)])
    plsc.subcore_barrier()
    pltpu.sync_copy(dst_hbm.at[pl.ds(wid * CH, CH)], idx_v)

    def body(j, carry):
        pltpu.sync_copy(ones_v, acc.at[idx_v.at[j]], add=True)
        return carry

    lax.fori_loop(0, CH, body, 0)
    plsc.subcore_barrier()
    pltpu.sync_copy(acc.at[pl.ds(s * RPT, RPT)],
                    out_hbm.at[pl.ds(c * ACC_ROWS + s * RPT, RPT)])


def _k2_body(x_ref, degt_ref, w_ref, g_ref):
    deg = degt_ref[:, 0:1] + degt_ref[:, 1:2] + 1.0
    dinv = lax.rsqrt(deg)
    h = jnp.dot(x_ref[...], w_ref[...], preferred_element_type=jnp.float32)
    g_ref[...] = h * dinv


def _k2_scale(x, deg_t, w):
    blk = 1000
    return pl.pallas_call(
        _k2_body,
        grid=(N // blk,),
        in_specs=[
            pl.BlockSpec((blk, D), lambda i: (i, 0)),
            pl.BlockSpec((blk, 2), lambda i: (i, 0)),
            pl.BlockSpec((D, D), lambda i: (0, 0)),
        ],
        out_specs=pl.BlockSpec((blk, D), lambda i: (i, 0)),
        out_shape=jax.ShapeDtypeStruct((N, D), jnp.float32),
    )(x, deg_t, w)


@functools.partial(
    pl.kernel,
    out_type=jax.ShapeDtypeStruct((NC, ACC_ROWS, D), jnp.float32),
    mesh=_mesh,
    scratch_types=[
        pltpu.VMEM((GRP, K), jnp.int32),
        pltpu.VMEM((GRP, K), jnp.int32),
        pltpu.VMEM((K, D), jnp.float32),
        pltpu.VMEM((K, D), jnp.float32),
        pltpu.VMEM((16, D), jnp.float32),
        pltpu.VMEM_SHARED((ACC_ROWS, D), jnp.float32),
        pltpu.SemaphoreType.DMA,
    ],
)
def _k3_edges(g_hbm, src_hbm, dst_hbm, out_hbm,
              idx_s, idx_d, rows0, rows1, zbuf, acc, sem):
    c = lax.axis_index("c")
    s = lax.axis_index("s")
    wid = s * NC + c
    for i in range(16):
        for j in range(D // 16):
            zbuf[i, pl.ds(j * 16, 16)] = jnp.zeros((16,), jnp.float32)
    for i in range(RPT // 16):
        pltpu.async_copy(zbuf, acc.at[pl.ds(s * RPT + i * 16, 16)], sem)
    for i in range(RPT // 16):
        pltpu.make_async_copy(zbuf, acc.at[pl.ds(s * RPT + i * 16, 16)],
                              sem).wait()
    plsc.subcore_barrier()

    def group(gi, carry):
        base = wid * CH + gi * GRP
        pltpu.sync_copy(src_hbm.at[pl.ds(base, GRP)], idx_s)
        pltpu.sync_copy(dst_hbm.at[pl.ds(base, GRP)], idx_d)
        pltpu.async_copy(g_hbm.at[idx_s.at[0]], rows0, sem)

        def pair(t, carry2):
            j = t * 2
            pltpu.make_async_copy(g_hbm.at[idx_s.at[j]], rows0, sem).wait()
            pltpu.async_copy(g_hbm.at[idx_s.at[j + 1]], rows1, sem)
            pltpu.sync_copy(rows0, acc.at[idx_d.at[j]], add=True)
            pltpu.make_async_copy(g_hbm.at[idx_s.at[j + 1]], rows1,
                                  sem).wait()

            @pl.when(j + 2 < GRP)
            def _():
                pltpu.async_copy(g_hbm.at[idx_s.at[j + 2]], rows0, sem)

            pltpu.sync_copy(rows1, acc.at[idx_d.at[j + 1]], add=True)
            return carry2

        lax.fori_loop(0, GRP // 2, pair, 0)
        return carry

    lax.fori_loop(0, CH // GRP, group, 0)
    plsc.subcore_barrier()
    pltpu.sync_copy(acc.at[pl.ds(s * RPT, RPT)],
                    out_hbm.at[c, pl.ds(s * RPT, RPT), :])


def _k4_body(part_ref, g_ref, degt_ref, b_ref, out_ref):
    deg = degt_ref[:, 0:1] + degt_ref[:, 1:2] + 1.0
    dinv = lax.rsqrt(deg)
    a = part_ref[0] + part_ref[1] + g_ref[...]
    out_ref[...] = jnp.maximum(a * dinv + b_ref[...], 0.0)


def _k4_combine(parts, g, deg_t, b2d):
    blk = 1000
    return pl.pallas_call(
        _k4_body,
        grid=(N // blk,),
        in_specs=[
            pl.BlockSpec((NC, blk, D), lambda i: (0, i, 0)),
            pl.BlockSpec((blk, D), lambda i: (i, 0)),
            pl.BlockSpec((blk, 2), lambda i: (i, 0)),
            pl.BlockSpec((1, D), lambda i: (0, 0)),
        ],
        out_specs=pl.BlockSpec((blk, D), lambda i: (i, 0)),
        out_shape=jax.ShapeDtypeStruct((N, D), jnp.float32),
    )(parts, g, deg_t, b2d)


def kernel(x, edge_index, W, b):
    ei = edge_index.astype(jnp.int32)
    pad = E_PAD - E
    src = jnp.concatenate([ei[0], jnp.zeros((pad,), jnp.int32)])
    dst = jnp.concatenate([ei[1], jnp.full((pad,), TRASH, jnp.int32)])
    src2d = src.reshape(ROWS_PAD, K)
    dst2d = dst.reshape(ROWS_PAD, K)

    deg_parts = _k1_deg(dst2d).reshape(NC, ACC_ROWS)
    deg_t = jnp.transpose(deg_parts)[:N]
    g = _k2_scale(x, deg_t, W)
    parts = _k3_edges(g, src2d, dst2d)
    b2d = b.reshape(1, D)
    return _k4_combine(parts, g, deg_t, b2d)
